# baseline (speedup 1.0000x reference)
import jax
import jax.numpy as jnp
from jax import lax
from jax.experimental import pallas as pl
from jax.experimental.pallas import tpu as pltpu
from jax.experimental.pallas import tpu_sc as plsc

_B = 16384
_D = 200
_NC = 2
_NS = 16
_NW = _NC * _NS
_BPW = _B // _NW
_K = 4
_M = _BPW // _K


def _gather_body(ent_hbm, rel_hbm, e1_idx_hbm, rel_idx_hbm, out_hbm,
                 idx_v, rows_v, sem):
    wid = lax.axis_index("s") * _NC + lax.axis_index("c")
    base = wid * _BPW

    pltpu.sync_copy(e1_idx_hbm.at[wid], idx_v)
    cps = [pltpu.async_copy(ent_hbm.at[idx_v.at[j]],
                            rows_v.at[pl.ds(j * _M, _M)], sem)
           for j in range(_K)]
    for c in cps:
        c.wait()
    pltpu.sync_copy(rows_v, out_hbm.at[pl.ds(base, _BPW), pl.ds(0, _D)])

    pltpu.sync_copy(rel_idx_hbm.at[wid], idx_v)
    cps = [pltpu.async_copy(rel_hbm.at[idx_v.at[j]],
                            rows_v.at[pl.ds(j * _M, _M)], sem)
           for j in range(_K)]
    for c in cps:
        c.wait()
    pltpu.sync_copy(rows_v, out_hbm.at[pl.ds(base, _BPW), pl.ds(_D, _D)])


def _gather(ent_table, rel_table, e1_idx, rel_idx):
    mesh = plsc.VectorSubcoreMesh(core_axis_name="c", subcore_axis_name="s")
    f = pl.kernel(
        _gather_body,
        mesh=mesh,
        out_type=jax.ShapeDtypeStruct((_B, 2 * _D), jnp.float32),
        scratch_types=[
            pltpu.VMEM((_K, _M), jnp.int32),
            pltpu.VMEM((_BPW, _D), jnp.float32),
            pltpu.SemaphoreType.DMA,
        ],
        compiler_params=pltpu.CompilerParams(use_tc_tiling_on_sc=False),
    )
    return f(ent_table, rel_table,
             e1_idx.reshape(_NW, _K, _M), rel_idx.reshape(_NW, _K, _M))


def kernel(ent_table, rel_table, e1_idx, rel_idx):
    out = _gather(ent_table, rel_table, e1_idx, rel_idx)
    return out.reshape(_B, 1, 20, 20)
